# baseline (speedup 1.0000x reference)
import jax
import jax.numpy as jnp
from jax.experimental import pallas as pl
from jax.experimental.pallas import tpu as pltpu

_TM = 1024


def _mean_body(w_ref, b_ref, wb_ref):
    D = w_ref.shape[1]
    wb_ref[:D, :] = (w_ref[0] + w_ref[1] + w_ref[2] + w_ref[3]) * 0.25
    bm = (b_ref[0] + b_ref[1] + b_ref[2] + b_ref[3]) * 0.25
    wb_ref[D:, :] = jnp.broadcast_to(bm, wb_ref[D:, :].shape)


def _gemm_body(x_ref, wb_ref, o_ref):
    D = x_ref.shape[1]
    o_ref[...] = (
        jnp.dot(
            x_ref[...],
            wb_ref[:D, :],
            preferred_element_type=jnp.float32,
        )
        + wb_ref[D, :]
    )


def kernel(x, W, b):
    T, D = x.shape
    R, _, E = W.shape
    wb = pl.pallas_call(
        _mean_body,
        in_specs=[
            pl.BlockSpec((R, D, E), lambda: (0, 0, 0)),
            pl.BlockSpec((R, E), lambda: (0, 0)),
        ],
        out_specs=pl.BlockSpec((D + 8, E), lambda: (0, 0)),
        out_shape=jax.ShapeDtypeStruct((D + 8, E), jnp.float32),
    )(W, b)
    return pl.pallas_call(
        _gemm_body,
        grid=(T // _TM,),
        in_specs=[
            pl.BlockSpec((_TM, D), lambda i: (i, 0)),
            pl.BlockSpec((D + 8, E), lambda i: (0, 0)),
        ],
        out_specs=pl.BlockSpec((_TM, E), lambda i: (i, 0)),
        out_shape=jax.ShapeDtypeStruct((T, E), jnp.float32),
        compiler_params=pltpu.CompilerParams(
            dimension_semantics=("arbitrary",),
        ),
    )(x, wb)


# speedup vs baseline: 1.0007x; 1.0007x over previous
import jax
import jax.numpy as jnp
from jax.experimental import pallas as pl
from jax.experimental.pallas import tpu as pltpu

_TM = 1024


def _mean_body(w_ref, b_ref, wm_ref, bm_ref):
    wm_ref[...] = (w_ref[0] + w_ref[1] + w_ref[2] + w_ref[3]) * 0.25
    bm = (b_ref[0] + b_ref[1] + b_ref[2] + b_ref[3]) * 0.25
    bm_ref[...] = jnp.broadcast_to(bm, bm_ref.shape)


def _gemm_body(x_ref, wm_ref, bm_ref, o_ref):
    o_ref[...] = (
        jnp.dot(x_ref[...], wm_ref[...], preferred_element_type=jnp.float32)
        + bm_ref[0, :]
    )


def kernel(x, W, b):
    T, D = x.shape
    R, _, E = W.shape
    wm, bm = pl.pallas_call(
        _mean_body,
        in_specs=[
            pl.BlockSpec((R, D, E), lambda: (0, 0, 0)),
            pl.BlockSpec((R, E), lambda: (0, 0)),
        ],
        out_specs=[
            pl.BlockSpec((D, E), lambda: (0, 0)),
            pl.BlockSpec((8, E), lambda: (0, 0)),
        ],
        out_shape=[
            jax.ShapeDtypeStruct((D, E), jnp.float32),
            jax.ShapeDtypeStruct((8, E), jnp.float32),
        ],
    )(W, b)
    return pl.pallas_call(
        _gemm_body,
        grid=(T // _TM,),
        in_specs=[
            pl.BlockSpec((_TM, D), lambda i: (i, 0)),
            pl.BlockSpec((D, E), lambda i: (0, 0)),
            pl.BlockSpec((8, E), lambda i: (0, 0)),
        ],
        out_specs=pl.BlockSpec((_TM, E), lambda i: (i, 0)),
        out_shape=jax.ShapeDtypeStruct((T, E), jnp.float32),
        compiler_params=pltpu.CompilerParams(
            dimension_semantics=("arbitrary",),
        ),
    )(x, wm, bm)


# speedup vs baseline: 1.0115x; 1.0108x over previous
import jax
import jax.numpy as jnp
from jax.experimental import pallas as pl
from jax.experimental.pallas import tpu as pltpu

_TM = 1024


def _outer(x_hbm, w_hbm, b_hbm, o_hbm, w_vmem, b_vmem, wm_ref, bm_ref,
           w_sem, b_sem):
    T, D = x_hbm.shape
    E = wm_ref.shape[1]
    cw = pltpu.make_async_copy(w_hbm, w_vmem, w_sem)
    cb = pltpu.make_async_copy(b_hbm, b_vmem, b_sem)
    cw.start()
    cb.start()
    cw.wait()
    cb.wait()
    wm_ref[...] = (w_vmem[0] + w_vmem[1] + w_vmem[2] + w_vmem[3]) * 0.25
    bm = (b_vmem[0] + b_vmem[1] + b_vmem[2] + b_vmem[3]) * 0.25
    bm_ref[...] = jnp.broadcast_to(bm, bm_ref.shape)

    def _inner(x_ref, o_ref):
        o_ref[...] = (
            jnp.dot(
                x_ref[...], wm_ref[...], preferred_element_type=jnp.float32
            )
            + bm_ref[0, :]
        )

    pltpu.emit_pipeline(
        _inner,
        grid=(T // _TM,),
        in_specs=[pl.BlockSpec((_TM, D), lambda i: (i, 0))],
        out_specs=[pl.BlockSpec((_TM, E), lambda i: (i, 0))],
    )(x_hbm, o_hbm)


def kernel(x, W, b):
    T, D = x.shape
    R, _, E = W.shape
    return pl.pallas_call(
        _outer,
        in_specs=[
            pl.BlockSpec(memory_space=pltpu.HBM),
            pl.BlockSpec(memory_space=pltpu.HBM),
            pl.BlockSpec(memory_space=pltpu.HBM),
        ],
        out_specs=pl.BlockSpec(memory_space=pltpu.HBM),
        out_shape=jax.ShapeDtypeStruct((T, E), jnp.float32),
        scratch_shapes=[
            pltpu.VMEM((R, D, E), jnp.float32),
            pltpu.VMEM((R, E), jnp.float32),
            pltpu.VMEM((D, E), jnp.float32),
            pltpu.VMEM((8, E), jnp.float32),
            pltpu.SemaphoreType.DMA,
            pltpu.SemaphoreType.DMA,
        ],
    )(x, W, b)
